# baseline (speedup 1.0000x reference)
import functools

import jax
import jax.numpy as jnp
from jax import lax
from jax.experimental import pallas as pl
from jax.experimental.pallas import tpu as pltpu
from jax.experimental.pallas import tpu_sc as plsc

DIM_H = 1024
NV = 320
GR = 2
DCODE = 128
TB = 512


def _tc_body(hs_ref, w0_ref, w1_ref, b0_ref, b1_ref, dist_ref, idx_ref):
    hs = hs_ref[...]
    l0 = jnp.dot(hs, w0_ref[...], preferred_element_type=jnp.float32) + b0_ref[...]
    l1 = jnp.dot(hs, w1_ref[...], preferred_element_type=jnp.float32) + b1_ref[...]
    iota = lax.broadcasted_iota(jnp.int32, (TB, NV), 1)
    big = jnp.int32(2 ** 30)

    m0 = jnp.max(l0, axis=1, keepdims=True)
    i0 = jnp.min(jnp.where(l0 == m0, iota, big), axis=1)
    m1 = jnp.max(l1, axis=1, keepdims=True)
    i1 = jnp.min(jnp.where(l1 == m1, iota, big), axis=1)

    dist_ref[:, 0, :] = (iota == i0[:, None]).astype(jnp.float32)
    dist_ref[:, 1, :] = (iota == i1[:, None]).astype(jnp.float32)
    idx_ref[:, 0] = i0
    idx_ref[:, 1] = i1 + NV


def _tc_call(hs2d, w0, w1, b0, b1):
    T = hs2d.shape[0]
    return pl.pallas_call(
        _tc_body,
        grid=(T // TB,),
        in_specs=[
            pl.BlockSpec((TB, DIM_H), lambda i: (i, 0)),
            pl.BlockSpec((DIM_H, NV), lambda i: (0, 0)),
            pl.BlockSpec((DIM_H, NV), lambda i: (0, 0)),
            pl.BlockSpec((1, NV), lambda i: (0, 0)),
            pl.BlockSpec((1, NV), lambda i: (0, 0)),
        ],
        out_specs=[
            pl.BlockSpec((TB, GR, NV), lambda i: (i, 0, 0)),
            pl.BlockSpec((TB, GR), lambda i: (i, 0)),
        ],
        out_shape=[
            jax.ShapeDtypeStruct((T, GR, NV), jnp.float32),
            jax.ShapeDtypeStruct((T, GR), jnp.int32),
        ],
    )(hs2d, w0, w1, b0, b1)


@functools.lru_cache(maxsize=None)
def _make_sc_gather(B, D):
    info = plsc.get_sparse_core_info()
    nw = info.num_cores * info.num_subcores
    b_per_w = B // nw
    mesh = plsc.VectorSubcoreMesh(core_axis_name="c", subcore_axis_name="s")

    @functools.partial(
        pl.kernel,
        mesh=mesh,
        out_type=jax.ShapeDtypeStruct((B, D), jnp.float32),
        scratch_types=[
            pltpu.VMEM((b_per_w,), jnp.int32),
            pltpu.VMEM((b_per_w, D), jnp.float32),
            pltpu.SemaphoreType.DMA,
        ],
    )
    def k(table_hbm, idx_hbm, out_hbm, idx_v, rows_v, sem):
        wid = lax.axis_index("s") * info.num_cores + lax.axis_index("c")
        base = wid * b_per_w
        pltpu.sync_copy(idx_hbm.at[pl.ds(base, b_per_w)], idx_v)
        pltpu.async_copy(table_hbm.at[idx_v], rows_v, sem).wait()
        pltpu.sync_copy(rows_v, out_hbm.at[pl.ds(base, b_per_w)])

    return k


def kernel(hidden_states, codevectors, w_proj, b_proj):
    B, S, H = hidden_states.shape
    T = B * S
    hs2d = hidden_states.reshape(T, H)
    w0 = w_proj[:, :NV]
    w1 = w_proj[:, NV:]
    b0 = b_proj[:NV].reshape(1, NV)
    b1 = b_proj[NV:].reshape(1, NV)
    dist, idx = _tc_call(hs2d, w0, w1, b0, b1)

    table = codevectors.reshape(GR * NV, DCODE)
    flat_idx = idx.reshape(T * GR)
    cv = _make_sc_gather(T * GR, DCODE)(table, flat_idx)
    cv = cv.reshape(B, S, GR * DCODE)
    return cv, dist


# speedup vs baseline: 1.4823x; 1.4823x over previous
import functools

import jax
import jax.numpy as jnp
from jax import lax
from jax.experimental import pallas as pl
from jax.experimental.pallas import tpu as pltpu
from jax.experimental.pallas import tpu_sc as plsc

DIM_H = 1024
NV = 320
GR = 2
DCODE = 128
TB = 512


def _tc_body(hs_ref, w0_ref, w1_ref, b0_ref, b1_ref, cb0_ref, cb1_ref, dist_ref, cv_ref):
    hs = hs_ref[...]
    l0 = jnp.dot(hs, w0_ref[...], preferred_element_type=jnp.float32) + b0_ref[...]
    l1 = jnp.dot(hs, w1_ref[...], preferred_element_type=jnp.float32) + b1_ref[...]
    iota = lax.broadcasted_iota(jnp.int32, (TB, NV), 1)
    big = jnp.int32(2 ** 30)

    m0 = jnp.max(l0, axis=1, keepdims=True)
    i0 = jnp.min(jnp.where(l0 == m0, iota, big), axis=1)
    m1 = jnp.max(l1, axis=1, keepdims=True)
    i1 = jnp.min(jnp.where(l1 == m1, iota, big), axis=1)

    oh0 = (iota == i0[:, None]).astype(jnp.float32)
    oh1 = (iota == i1[:, None]).astype(jnp.float32)
    dist_ref[:, 0, :] = oh0
    dist_ref[:, 1, :] = oh1
    cv0 = jnp.dot(oh0, cb0_ref[...], preferred_element_type=jnp.float32)
    cv1 = jnp.dot(oh1, cb1_ref[...], preferred_element_type=jnp.float32)
    cv_ref[:, :DCODE] = cv0
    cv_ref[:, DCODE:] = cv1


def _tc_call(hs2d, w0, w1, b0, b1, cb0, cb1):
    T = hs2d.shape[0]
    return pl.pallas_call(
        _tc_body,
        grid=(T // TB,),
        in_specs=[
            pl.BlockSpec((TB, DIM_H), lambda i: (i, 0)),
            pl.BlockSpec((DIM_H, NV), lambda i: (0, 0)),
            pl.BlockSpec((DIM_H, NV), lambda i: (0, 0)),
            pl.BlockSpec((1, NV), lambda i: (0, 0)),
            pl.BlockSpec((1, NV), lambda i: (0, 0)),
            pl.BlockSpec((NV, DCODE), lambda i: (0, 0)),
            pl.BlockSpec((NV, DCODE), lambda i: (0, 0)),
        ],
        out_specs=[
            pl.BlockSpec((TB, GR, NV), lambda i: (i, 0, 0)),
            pl.BlockSpec((TB, GR * DCODE), lambda i: (i, 0)),
        ],
        out_shape=[
            jax.ShapeDtypeStruct((T, GR, NV), jnp.float32),
            jax.ShapeDtypeStruct((T, GR * DCODE), jnp.float32),
        ],
    )(hs2d, w0, w1, b0, b1, cb0, cb1)


@functools.lru_cache(maxsize=None)
def _make_sc_gather(B, D):
    info = plsc.get_sparse_core_info()
    nw = info.num_cores * info.num_subcores
    b_per_w = B // nw
    mesh = plsc.VectorSubcoreMesh(core_axis_name="c", subcore_axis_name="s")

    @functools.partial(
        pl.kernel,
        mesh=mesh,
        out_type=jax.ShapeDtypeStruct((B, D), jnp.float32),
        scratch_types=[
            pltpu.VMEM((b_per_w,), jnp.int32),
            pltpu.VMEM((b_per_w, D), jnp.float32),
            pltpu.SemaphoreType.DMA,
        ],
    )
    def k(table_hbm, idx_hbm, out_hbm, idx_v, rows_v, sem):
        wid = lax.axis_index("s") * info.num_cores + lax.axis_index("c")
        base = wid * b_per_w
        pltpu.sync_copy(idx_hbm.at[pl.ds(base, b_per_w)], idx_v)
        pltpu.async_copy(table_hbm.at[idx_v], rows_v, sem).wait()
        pltpu.sync_copy(rows_v, out_hbm.at[pl.ds(base, b_per_w)])

    return k


def kernel(hidden_states, codevectors, w_proj, b_proj):
    B, S, H = hidden_states.shape
    T = B * S
    hs2d = hidden_states.reshape(T, H)
    w0 = w_proj[:, :NV]
    w1 = w_proj[:, NV:]
    b0 = b_proj[:NV].reshape(1, NV)
    b1 = b_proj[NV:].reshape(1, NV)
    cb = codevectors.reshape(GR, NV, DCODE)
    dist, cv = _tc_call(hs2d, w0, w1, b0, b1, cb[0], cb[1])
    cv = cv.reshape(B, S, GR * DCODE)
    return cv, dist


# speedup vs baseline: 1.5136x; 1.0211x over previous
import functools

import jax
import jax.numpy as jnp
from jax import lax
from jax.experimental import pallas as pl
from jax.experimental.pallas import tpu as pltpu
from jax.experimental.pallas import tpu_sc as plsc

DIM_H = 1024
NV = 320
GR = 2
DCODE = 128
TB = 1024


def _tc_body(hs_ref, w0_ref, w1_ref, b0_ref, b1_ref, cb0_ref, cb1_ref, dist_ref, cv_ref):
    hs = hs_ref[...]
    l0 = jnp.dot(hs, w0_ref[...], preferred_element_type=jnp.float32) + b0_ref[...]
    l1 = jnp.dot(hs, w1_ref[...], preferred_element_type=jnp.float32) + b1_ref[...]
    iota = lax.broadcasted_iota(jnp.int32, (TB, NV), 1)
    big = jnp.int32(2 ** 30)

    m0 = jnp.max(l0, axis=1, keepdims=True)
    i0 = jnp.min(jnp.where(l0 == m0, iota, big), axis=1)
    m1 = jnp.max(l1, axis=1, keepdims=True)
    i1 = jnp.min(jnp.where(l1 == m1, iota, big), axis=1)

    oh0 = (iota == i0[:, None]).astype(jnp.float32)
    oh1 = (iota == i1[:, None]).astype(jnp.float32)
    dist_ref[:, 0, :] = oh0
    dist_ref[:, 1, :] = oh1
    cv0 = jnp.dot(oh0, cb0_ref[...], preferred_element_type=jnp.float32)
    cv1 = jnp.dot(oh1, cb1_ref[...], preferred_element_type=jnp.float32)
    cv_ref[:, :DCODE] = cv0
    cv_ref[:, DCODE:] = cv1


def _tc_call(hs2d, w0, w1, b0, b1, cb0, cb1):
    T = hs2d.shape[0]
    return pl.pallas_call(
        _tc_body,
        grid=(T // TB,),
        in_specs=[
            pl.BlockSpec((TB, DIM_H), lambda i: (i, 0)),
            pl.BlockSpec((DIM_H, NV), lambda i: (0, 0)),
            pl.BlockSpec((DIM_H, NV), lambda i: (0, 0)),
            pl.BlockSpec((1, NV), lambda i: (0, 0)),
            pl.BlockSpec((1, NV), lambda i: (0, 0)),
            pl.BlockSpec((NV, DCODE), lambda i: (0, 0)),
            pl.BlockSpec((NV, DCODE), lambda i: (0, 0)),
        ],
        out_specs=[
            pl.BlockSpec((TB, GR, NV), lambda i: (i, 0, 0)),
            pl.BlockSpec((TB, GR * DCODE), lambda i: (i, 0)),
        ],
        out_shape=[
            jax.ShapeDtypeStruct((T, GR, NV), jnp.float32),
            jax.ShapeDtypeStruct((T, GR * DCODE), jnp.float32),
        ],
    )(hs2d, w0, w1, b0, b1, cb0, cb1)


@functools.lru_cache(maxsize=None)
def _make_sc_gather(B, D):
    info = plsc.get_sparse_core_info()
    nw = info.num_cores * info.num_subcores
    b_per_w = B // nw
    mesh = plsc.VectorSubcoreMesh(core_axis_name="c", subcore_axis_name="s")

    @functools.partial(
        pl.kernel,
        mesh=mesh,
        out_type=jax.ShapeDtypeStruct((B, D), jnp.float32),
        scratch_types=[
            pltpu.VMEM((b_per_w,), jnp.int32),
            pltpu.VMEM((b_per_w, D), jnp.float32),
            pltpu.SemaphoreType.DMA,
        ],
    )
    def k(table_hbm, idx_hbm, out_hbm, idx_v, rows_v, sem):
        wid = lax.axis_index("s") * info.num_cores + lax.axis_index("c")
        base = wid * b_per_w
        pltpu.sync_copy(idx_hbm.at[pl.ds(base, b_per_w)], idx_v)
        pltpu.async_copy(table_hbm.at[idx_v], rows_v, sem).wait()
        pltpu.sync_copy(rows_v, out_hbm.at[pl.ds(base, b_per_w)])

    return k


def kernel(hidden_states, codevectors, w_proj, b_proj):
    B, S, H = hidden_states.shape
    T = B * S
    hs2d = hidden_states.reshape(T, H)
    w0 = w_proj[:, :NV]
    w1 = w_proj[:, NV:]
    b0 = b_proj[:NV].reshape(1, NV)
    b1 = b_proj[NV:].reshape(1, NV)
    cb = codevectors.reshape(GR, NV, DCODE)
    dist, cv = _tc_call(hs2d, w0, w1, b0, b1, cb[0], cb[1])
    cv = cv.reshape(B, S, GR * DCODE)
    return cv, dist
